# baseline (speedup 1.0000x reference)
import functools

import jax
import jax.numpy as jnp
from jax.experimental import pallas as pl
from jax.experimental.pallas import tpu as pltpu

N = 10000
IN_CH = 256
D = 32
SD = 3
K = 40


def _bn(x, gamma, beta, eps=1e-5):
    mu = jnp.mean(x, axis=0)
    var = jnp.var(x, axis=0)
    return gamma * (x - mu) / jnp.sqrt(var + eps) + beta


def _elu(v):
    return jnp.where(v > 0, v, jnp.exp(jnp.minimum(v, 0.0)) - 1.0)


def _pre_mlp_body(x_ref, w1_ref, b1_ref, w2_ref, b2_ref, o_ref):
    h = _elu(x_ref[...] @ w1_ref[...] + b1_ref[...])
    h = _elu(h @ w2_ref[...] + b2_ref[...])
    o_ref[...] = h


def _pre_mlp(x, W1, b1, W2, b2):
    R = 1000
    grid = (N // R,)
    return pl.pallas_call(
        _pre_mlp_body,
        grid=grid,
        in_specs=[
            pl.BlockSpec((R, IN_CH), lambda i: (i, 0)),
            pl.BlockSpec((IN_CH, D), lambda i: (0, 0)),
            pl.BlockSpec((1, D), lambda i: (0, 0)),
            pl.BlockSpec((D, D), lambda i: (0, 0)),
            pl.BlockSpec((1, D), lambda i: (0, 0)),
        ],
        out_specs=pl.BlockSpec((R, D), lambda i: (i, 0)),
        out_shape=jax.ShapeDtypeStruct((N, D), jnp.float32),
    )(x, W1, b1[None, :], W2, b2[None, :])


def kernel(x, batch, original_coords, step_count, num_layer,
           W_pre1, b_pre1, W_pre2, b_pre2, g1, be1,
           W_s, W_h, b_h, W_lin, b_lin,
           W_post1, b_post1, W_post2, b_post2, g2, be2):
    h = _pre_mlp(x, W_pre1, b_pre1, W_pre2, b_pre2)
    h = _bn(h, g1, be1)
    x_input = h
    h_l = h @ W_h + b_h
    s_l = h @ W_s
    ss = jnp.sum(s_l * s_l, axis=-1)
    d2 = ss[:, None] + ss[None, :] - 2.0 * (s_l @ s_l.T)
    ar = jnp.arange(N)
    d2 = d2.at[ar, ar].set(jnp.inf)
    _, nn_idx = jax.lax.top_k(-d2, K)
    s_nbr = s_l[nn_idx]
    d2_nn = jnp.sum((s_l[:, None, :] - s_nbr) ** 2, axis=-1)
    potential = jnp.exp(-d2_nn)
    h_nbr = h_l[nn_idx]
    feat = potential[..., None] * h_nbr
    mean_agg = jnp.mean(feat, axis=1)
    max_agg = jnp.max(feat, axis=1)
    agg = jnp.concatenate([mean_agg, max_agg], axis=-1)
    xgn = jnp.concatenate([agg, h], axis=-1) @ W_lin + b_lin
    z = jnp.concatenate([xgn, s_l, x_input], axis=-1)
    z = jax.nn.elu(z @ W_post1 + b_post1)
    z = jax.nn.elu(z @ W_post2 + b_post2)
    z = _bn(z, g2, be2)
    return z


# speedup vs baseline: 3.9846x; 3.9846x over previous
import jax
import jax.numpy as jnp
from jax.experimental import pallas as pl
from jax.experimental.pallas import tpu as pltpu

N = 10000
IN_CH = 256
D = 32
SD = 3
K = 40

NPAD = 10240
R = 128
NSTRIPS = NPAD // R

_BIG_IDX = 2**30


def _dot(a, b):
    return jax.lax.dot(a, b)


def _elu(v):
    return jnp.where(v > 0, v, jnp.exp(jnp.minimum(v, 0.0)) - 1.0)


def _bn(x, gamma, beta, eps=1e-5):
    mu = jnp.mean(x, axis=0, keepdims=True)
    var = jnp.mean((x - mu) ** 2, axis=0, keepdims=True)
    return gamma * (x - mu) / jnp.sqrt(var + eps) + beta


def _k1_body(x_ref, w1_ref, b1_ref, w2_ref, b2_ref, g1_ref, be1_ref,
             wh_ref, bh_ref, ws_ref, h_ref, hl_ref, srow_ref):
    h = _elu(_dot(x_ref[...], w1_ref[...]) + b1_ref[...])
    h = _elu(_dot(h, w2_ref[...]) + b2_ref[...])
    h = _bn(h, g1_ref[...], be1_ref[...])
    h_ref[...] = h
    hl_ref[...] = _dot(h, wh_ref[...]) + bh_ref[...]
    s = _dot(h, ws_ref[...])
    ss = jnp.sum(s * s, axis=1, keepdims=True)
    srow_ref[...] = jnp.concatenate(
        [s, ss, jnp.zeros((s.shape[0], 4), jnp.float32)], axis=1)


def _k1(x, W1, b1, W2, b2, g1, be1, Wh, bh, Ws):
    return pl.pallas_call(
        _k1_body,
        out_shape=(
            jax.ShapeDtypeStruct((N, D), jnp.float32),
            jax.ShapeDtypeStruct((N, D), jnp.float32),
            jax.ShapeDtypeStruct((N, 8), jnp.float32),
        ),
    )(x, W1, b1[None, :], W2, b2[None, :], g1[None, :], be1[None, :],
      Wh, bh[None, :], Ws)


def _k2_body(srow_ref, scol_ref, idx_ref, d2_ref):
    i = pl.program_id(0)
    ssr = srow_ref[:, 3:4]
    ssc = scol_ref[3:4, :]
    g = _dot(srow_ref[:, 0:3], scol_ref[0:3, :])
    d2 = (ssr + ssc) - 2.0 * g
    cid = jax.lax.broadcasted_iota(jnp.int32, (R, NPAD), 1)
    rid = i * R + jax.lax.broadcasted_iota(jnp.int32, (R, NPAD), 0)
    d2_ref[...] = jnp.where((cid >= N) | (cid == rid), jnp.inf, d2)
    lk = jax.lax.broadcasted_iota(jnp.int32, (R, K), 1)

    def extract(k, idxacc):
        d = d2_ref[...]
        m = jnp.min(d, axis=1, keepdims=True)
        eq = d <= m
        c = jax.lax.broadcasted_iota(jnp.int32, (R, NPAD), 1)
        idxv = jnp.min(jnp.where(eq, c, _BIG_IDX), axis=1, keepdims=True)
        d2_ref[...] = jnp.where(eq, jnp.inf, d)
        return jnp.where(lk == k, idxv, idxacc)

    idx_ref[...] = jax.lax.fori_loop(
        0, K, extract, jnp.zeros((R, K), jnp.int32))


def _k2(srow_pad, scol_pad):
    return pl.pallas_call(
        _k2_body,
        grid=(NSTRIPS,),
        in_specs=[
            pl.BlockSpec((R, 8), lambda i: (i, 0)),
            pl.BlockSpec((8, NPAD), lambda i: (0, 0)),
        ],
        out_specs=pl.BlockSpec((R, K), lambda i: (i, 0)),
        out_shape=jax.ShapeDtypeStruct((NPAD, K), jnp.int32),
        scratch_shapes=[pltpu.VMEM((R, NPAD), jnp.float32)],
    )(srow_pad, scol_pad)


def _k4_body(agg_ref, h_ref, srow_ref, wla_ref, wlh_ref, bl_ref,
             wpx_ref, wps_ref, wph_ref, bp1_ref, wp2_ref, bp2_ref,
             g2_ref, be2_ref, o_ref):
    h = h_ref[...]
    xgn = (_dot(agg_ref[...], wla_ref[...]) + _dot(h, wlh_ref[...])
           + bl_ref[...])
    zs = (srow_ref[:, 0:1] * wps_ref[0:1, :]
          + srow_ref[:, 1:2] * wps_ref[1:2, :]
          + srow_ref[:, 2:3] * wps_ref[2:3, :])
    z = _elu(_dot(xgn, wpx_ref[...]) + zs + _dot(h, wph_ref[...])
             + bp1_ref[...])
    z = _elu(_dot(z, wp2_ref[...]) + bp2_ref[...])
    o_ref[...] = _bn(z, g2_ref[...], be2_ref[...])


def _k4(agg, h, srow, W_lin, b_lin, W_post1, b_post1, W_post2, b_post2,
        g2, be2):
    return pl.pallas_call(
        _k4_body,
        out_shape=jax.ShapeDtypeStruct((N, D), jnp.float32),
    )(agg, h, srow, W_lin[:2 * D], W_lin[2 * D:], b_lin[None, :],
      W_post1[:D], W_post1[D:D + SD], W_post1[D + SD:], b_post1[None, :],
      W_post2, b_post2[None, :], g2[None, :], be2[None, :])


def kernel(x, batch, original_coords, step_count, num_layer,
           W_pre1, b_pre1, W_pre2, b_pre2, g1, be1,
           W_s, W_h, b_h, W_lin, b_lin,
           W_post1, b_post1, W_post2, b_post2, g2, be2):
    h, h_l, srow = _k1(x, W_pre1, b_pre1, W_pre2, b_pre2, g1, be1,
                       W_h, b_h, W_s)
    srow_pad = jnp.pad(srow, ((0, NPAD - N), (0, 0)))
    scol_pad = srow_pad.T
    nn_idx = _k2(srow_pad, scol_pad)[:N]
    s_l = srow[:, :SD]
    s_nbr = s_l[nn_idx]
    d2_nn = jnp.sum((s_l[:, None, :] - s_nbr) ** 2, axis=-1)
    p = jnp.exp(-d2_nn)
    h_nbr = h_l[nn_idx]
    feat = p[..., None] * h_nbr
    agg = jnp.concatenate([jnp.mean(feat, axis=1), jnp.max(feat, axis=1)],
                          axis=-1)
    return _k4(agg, h, srow, W_lin, b_lin, W_post1, b_post1,
               W_post2, b_post2, g2, be2)


# speedup vs baseline: 4.0110x; 1.0066x over previous
import functools

import jax
import jax.numpy as jnp
from jax.experimental import pallas as pl
from jax.experimental.pallas import tpu as pltpu
from jax.experimental.pallas import tpu_sc as plsc

N = 10000
IN_CH = 256
D = 32
SD = 3
K = 40

NPAD = 10240
R = 128
NSTRIPS = NPAD // R

_BIG_IDX = 2**30


def _dot(a, b):
    return jax.lax.dot(a, b)


def _elu(v):
    return jnp.where(v > 0, v, jnp.exp(jnp.minimum(v, 0.0)) - 1.0)


def _bn(x, gamma, beta, eps=1e-5):
    mu = jnp.mean(x, axis=0, keepdims=True)
    var = jnp.mean((x - mu) ** 2, axis=0, keepdims=True)
    return gamma * (x - mu) / jnp.sqrt(var + eps) + beta


def _k1_body(x_ref, w1_ref, b1_ref, w2_ref, b2_ref, g1_ref, be1_ref,
             wh_ref, bh_ref, ws_ref, h_ref, hl_ref, srow_ref):
    h = _elu(_dot(x_ref[...], w1_ref[...]) + b1_ref[...])
    h = _elu(_dot(h, w2_ref[...]) + b2_ref[...])
    h = _bn(h, g1_ref[...], be1_ref[...])
    h_ref[...] = h
    hl_ref[...] = _dot(h, wh_ref[...]) + bh_ref[...]
    s = _dot(h, ws_ref[...])
    ss = jnp.sum(s * s, axis=1, keepdims=True)
    srow_ref[...] = jnp.concatenate(
        [s, ss, jnp.zeros((s.shape[0], 4), jnp.float32)], axis=1)


def _k1(x, W1, b1, W2, b2, g1, be1, Wh, bh, Ws):
    return pl.pallas_call(
        _k1_body,
        out_shape=(
            jax.ShapeDtypeStruct((N, D), jnp.float32),
            jax.ShapeDtypeStruct((N, D), jnp.float32),
            jax.ShapeDtypeStruct((N, 8), jnp.float32),
        ),
    )(x, W1, b1[None, :], W2, b2[None, :], g1[None, :], be1[None, :],
      Wh, bh[None, :], Ws)


def _k2_body(srow_ref, scol_ref, idx_ref, d2_ref):
    i = pl.program_id(0)
    ssr = srow_ref[:, 3:4]
    ssc = scol_ref[3:4, :]
    g = _dot(srow_ref[:, 0:3], scol_ref[0:3, :])
    d2 = (ssr + ssc) - 2.0 * g
    cid = jax.lax.broadcasted_iota(jnp.int32, (R, NPAD), 1)
    rid = i * R + jax.lax.broadcasted_iota(jnp.int32, (R, NPAD), 0)
    d2_ref[...] = jnp.where((cid >= N) | (cid == rid), jnp.inf, d2)
    lk = jax.lax.broadcasted_iota(jnp.int32, (R, K), 1)

    def extract(k, idxacc):
        d = d2_ref[...]
        m = jnp.min(d, axis=1, keepdims=True)
        eq = d <= m
        c = jax.lax.broadcasted_iota(jnp.int32, (R, NPAD), 1)
        idxv = jnp.min(jnp.where(eq, c, _BIG_IDX), axis=1, keepdims=True)
        d2_ref[...] = jnp.where(eq, jnp.inf, d)
        return jnp.where(lk == k, idxv, idxacc)

    idx_ref[...] = jax.lax.fori_loop(
        0, K, extract, jnp.zeros((R, K), jnp.int32))


def _k2(srow_pad, scol_pad):
    return pl.pallas_call(
        _k2_body,
        grid=(NSTRIPS,),
        in_specs=[
            pl.BlockSpec((R, 8), lambda i: (i, 0)),
            pl.BlockSpec((8, NPAD), lambda i: (0, 0)),
        ],
        out_specs=pl.BlockSpec((R, K), lambda i: (i, 0)),
        out_shape=jax.ShapeDtypeStruct((NPAD, K), jnp.int32),
        scratch_shapes=[pltpu.VMEM((R, NPAD), jnp.float32)],
    )(srow_pad, scol_pad)


NW = 32
DPW = NPAD // NW
KP = 48


def _k3_body(idx_hbm, p_hbm, h_hbm, out_hbm,
             idx_v, p_all, rows_v, out_v, sem):
    wid = jax.lax.axis_index("s") * 2 + jax.lax.axis_index("c")
    base = wid * DPW
    pltpu.sync_copy(idx_hbm.at[pl.ds(base, DPW)], idx_v)
    pltpu.sync_copy(p_hbm.at[pl.ds(base, DPW)], p_all)

    def per_dst(d, carry):
        pltpu.async_copy(h_hbm.at[idx_v.at[d]], rows_v, sem).wait()
        dd = jnp.full((16,), d, jnp.int32)
        acc0 = jnp.zeros((16,), jnp.float32)
        acc1 = jnp.zeros((16,), jnp.float32)
        mx0 = jnp.full((16,), -jnp.inf, jnp.float32)
        mx1 = jnp.full((16,), -jnp.inf, jnp.float32)
        for k in range(K):
            pk = plsc.load_gather(p_all, [dd, jnp.full((16,), k, jnp.int32)])
            t0 = pk * rows_v[k, 0:16]
            t1 = pk * rows_v[k, 16:32]
            acc0 = acc0 + t0
            acc1 = acc1 + t1
            mx0 = jnp.maximum(mx0, t0)
            mx1 = jnp.maximum(mx1, t1)
        out_v[d, 0:16] = acc0 * (1.0 / K)
        out_v[d, 16:32] = acc1 * (1.0 / K)
        out_v[d, 32:48] = mx0
        out_v[d, 48:64] = mx1
        return carry

    jax.lax.fori_loop(0, DPW, per_dst, 0)
    pltpu.sync_copy(out_v, out_hbm.at[pl.ds(base, DPW)])


def _k3(idx48, p48, h_pad):
    mesh = plsc.VectorSubcoreMesh(core_axis_name="c", subcore_axis_name="s")
    kfn = functools.partial(
        pl.kernel, mesh=mesh,
        compiler_params=pltpu.CompilerParams(
            needs_layout_passes=False, use_tc_tiling_on_sc=False),
        out_type=jax.ShapeDtypeStruct((NPAD, 2 * D), jnp.float32),
        scratch_types=[
            pltpu.VMEM((DPW, KP), jnp.int32),
            pltpu.VMEM((DPW, KP), jnp.float32),
            pltpu.VMEM((KP, D), jnp.float32),
            pltpu.VMEM((DPW, 2 * D), jnp.float32),
            pltpu.SemaphoreType.DMA,
        ],
    )(_k3_body)
    return kfn(idx48, p48, h_pad)


def _k4_body(agg_ref, h_ref, srow_ref, wla_ref, wlh_ref, bl_ref,
             wpx_ref, wps_ref, wph_ref, bp1_ref, wp2_ref, bp2_ref,
             g2_ref, be2_ref, o_ref):
    h = h_ref[...]
    xgn = (_dot(agg_ref[...], wla_ref[...]) + _dot(h, wlh_ref[...])
           + bl_ref[...])
    zs = (srow_ref[:, 0:1] * wps_ref[0:1, :]
          + srow_ref[:, 1:2] * wps_ref[1:2, :]
          + srow_ref[:, 2:3] * wps_ref[2:3, :])
    z = _elu(_dot(xgn, wpx_ref[...]) + zs + _dot(h, wph_ref[...])
             + bp1_ref[...])
    z = _elu(_dot(z, wp2_ref[...]) + bp2_ref[...])
    o_ref[...] = _bn(z, g2_ref[...], be2_ref[...])


def _k4(agg, h, srow, W_lin, b_lin, W_post1, b_post1, W_post2, b_post2,
        g2, be2):
    return pl.pallas_call(
        _k4_body,
        out_shape=jax.ShapeDtypeStruct((N, D), jnp.float32),
    )(agg, h, srow, W_lin[:2 * D], W_lin[2 * D:], b_lin[None, :],
      W_post1[:D], W_post1[D:D + SD], W_post1[D + SD:], b_post1[None, :],
      W_post2, b_post2[None, :], g2[None, :], be2[None, :])


def kernel(x, batch, original_coords, step_count, num_layer,
           W_pre1, b_pre1, W_pre2, b_pre2, g1, be1,
           W_s, W_h, b_h, W_lin, b_lin,
           W_post1, b_post1, W_post2, b_post2, g2, be2):
    h, h_l, srow = _k1(x, W_pre1, b_pre1, W_pre2, b_pre2, g1, be1,
                       W_h, b_h, W_s)
    srow_pad = jnp.pad(srow, ((0, NPAD - N), (0, 0)))
    scol_pad = srow_pad.T
    nn_idx = _k2(srow_pad, scol_pad)
    idx48 = jnp.pad(nn_idx, ((0, 0), (0, KP - K)))
    h_l_pad = jnp.pad(h_l, ((0, NPAD - N), (0, 0)))
    s_l = srow[:, :SD]
    s_nbr = s_l[nn_idx[:N]]
    d2_nn = jnp.sum((s_l[:, None, :] - s_nbr) ** 2, axis=-1)
    p = jnp.exp(-d2_nn)
    p48 = jnp.pad(p, ((0, NPAD - N), (0, KP - K)))
    agg = _k3(idx48, p48, h_l_pad)[:N]
    return _k4(agg, h, srow, W_lin, b_lin, W_post1, b_post1,
               W_post2, b_post2, g2, be2)


# speedup vs baseline: 4.0436x; 1.0081x over previous
import functools

import jax
import jax.numpy as jnp
from jax.experimental import pallas as pl
from jax.experimental.pallas import tpu as pltpu
from jax.experimental.pallas import tpu_sc as plsc

N = 10000
IN_CH = 256
D = 32
SD = 3
K = 40

NPAD = 10240
R = 256
NSTRIPS = NPAD // R

_BIG_IDX = 2**30


def _dot(a, b):
    return jax.lax.dot(a, b)


def _elu(v):
    return jnp.where(v > 0, v, jnp.exp(jnp.minimum(v, 0.0)) - 1.0)


def _bn(x, gamma, beta, eps=1e-5):
    mu = jnp.mean(x, axis=0, keepdims=True)
    var = jnp.mean((x - mu) ** 2, axis=0, keepdims=True)
    return gamma * (x - mu) / jnp.sqrt(var + eps) + beta


def _k1_body(x_ref, w1_ref, b1_ref, w2_ref, b2_ref, g1_ref, be1_ref,
             wh_ref, bh_ref, ws_ref, h_ref, hl_ref, srow_ref):
    h = _elu(_dot(x_ref[...], w1_ref[...]) + b1_ref[...])
    h = _elu(_dot(h, w2_ref[...]) + b2_ref[...])
    h = _bn(h, g1_ref[...], be1_ref[...])
    h_ref[...] = h
    hl_ref[...] = _dot(h, wh_ref[...]) + bh_ref[...]
    s = _dot(h, ws_ref[...])
    ss = jnp.sum(s * s, axis=1, keepdims=True)
    srow_ref[...] = jnp.concatenate(
        [s, ss, jnp.zeros((s.shape[0], 4), jnp.float32)], axis=1)


def _k1(x, W1, b1, W2, b2, g1, be1, Wh, bh, Ws):
    return pl.pallas_call(
        _k1_body,
        out_shape=(
            jax.ShapeDtypeStruct((N, D), jnp.float32),
            jax.ShapeDtypeStruct((N, D), jnp.float32),
            jax.ShapeDtypeStruct((N, 8), jnp.float32),
        ),
    )(x, W1, b1[None, :], W2, b2[None, :], g1[None, :], be1[None, :],
      Wh, bh[None, :], Ws)


def _k2_body(srow_ref, scol_ref, idx_ref, d2_ref):
    i = pl.program_id(0)
    ssr = srow_ref[:, 3:4]
    ssc = scol_ref[3:4, :]
    g = _dot(srow_ref[:, 0:3], scol_ref[0:3, :])
    d2 = (ssr + ssc) - 2.0 * g
    cid = jax.lax.broadcasted_iota(jnp.int32, (R, NPAD), 1)
    rid = i * R + jax.lax.broadcasted_iota(jnp.int32, (R, NPAD), 0)
    d2_ref[...] = jnp.where((cid >= N) | (cid == rid), jnp.inf, d2)
    lk = jax.lax.broadcasted_iota(jnp.int32, (R, K), 1)

    def extract(k, idxacc):
        d = d2_ref[...]
        m = jnp.min(d, axis=1, keepdims=True)
        eq = d <= m
        c = jax.lax.broadcasted_iota(jnp.int32, (R, NPAD), 1)
        idxv = jnp.min(jnp.where(eq, c, _BIG_IDX), axis=1, keepdims=True)
        d2_ref[...] = jnp.where(eq, jnp.inf, d)
        return jnp.where(lk == k, idxv, idxacc)

    idx_ref[...] = jax.lax.fori_loop(
        0, K, extract, jnp.zeros((R, K), jnp.int32))


def _k2(srow_pad, scol_pad):
    return pl.pallas_call(
        _k2_body,
        grid=(NSTRIPS,),
        in_specs=[
            pl.BlockSpec((R, 8), lambda i: (i, 0)),
            pl.BlockSpec((8, NPAD), lambda i: (0, 0)),
        ],
        out_specs=pl.BlockSpec((R, K), lambda i: (i, 0)),
        out_shape=jax.ShapeDtypeStruct((NPAD, K), jnp.int32),
        scratch_shapes=[pltpu.VMEM((R, NPAD), jnp.float32)],
    )(srow_pad, scol_pad)


NW = 32
DPW = NPAD // NW
KP = 48


def _k3_body(idx_hbm, p_hbm, h_hbm, out_hbm,
             idx_v, p_all, rows_v, out_v, sem):
    wid = jax.lax.axis_index("s") * 2 + jax.lax.axis_index("c")
    base = wid * DPW
    pltpu.sync_copy(idx_hbm.at[pl.ds(base, DPW)], idx_v)
    pltpu.sync_copy(p_hbm.at[pl.ds(base, DPW)], p_all)

    def per_dst(d, carry):
        pltpu.async_copy(h_hbm.at[idx_v.at[d]], rows_v, sem).wait()
        dd = jnp.full((16,), d, jnp.int32)
        acc0 = jnp.zeros((16,), jnp.float32)
        acc1 = jnp.zeros((16,), jnp.float32)
        mx0 = jnp.full((16,), -jnp.inf, jnp.float32)
        mx1 = jnp.full((16,), -jnp.inf, jnp.float32)
        for k in range(K):
            pk = plsc.load_gather(p_all, [dd, jnp.full((16,), k, jnp.int32)])
            t0 = pk * rows_v[k, 0:16]
            t1 = pk * rows_v[k, 16:32]
            acc0 = acc0 + t0
            acc1 = acc1 + t1
            mx0 = jnp.maximum(mx0, t0)
            mx1 = jnp.maximum(mx1, t1)
        out_v[d, 0:16] = acc0 * (1.0 / K)
        out_v[d, 16:32] = acc1 * (1.0 / K)
        out_v[d, 32:48] = mx0
        out_v[d, 48:64] = mx1
        return carry

    jax.lax.fori_loop(0, DPW, per_dst, 0)
    pltpu.sync_copy(out_v, out_hbm.at[pl.ds(base, DPW)])


def _k3(idx48, p48, h_pad):
    mesh = plsc.VectorSubcoreMesh(core_axis_name="c", subcore_axis_name="s")
    kfn = functools.partial(
        pl.kernel, mesh=mesh,
        compiler_params=pltpu.CompilerParams(
            needs_layout_passes=False, use_tc_tiling_on_sc=False),
        out_type=jax.ShapeDtypeStruct((NPAD, 2 * D), jnp.float32),
        scratch_types=[
            pltpu.VMEM((DPW, KP), jnp.int32),
            pltpu.VMEM((DPW, KP), jnp.float32),
            pltpu.VMEM((KP, D), jnp.float32),
            pltpu.VMEM((DPW, 2 * D), jnp.float32),
            pltpu.SemaphoreType.DMA,
        ],
    )(_k3_body)
    return kfn(idx48, p48, h_pad)


def _k4_body(agg_ref, h_ref, srow_ref, wla_ref, wlh_ref, bl_ref,
             wpx_ref, wps_ref, wph_ref, bp1_ref, wp2_ref, bp2_ref,
             g2_ref, be2_ref, o_ref):
    h = h_ref[...]
    xgn = (_dot(agg_ref[...], wla_ref[...]) + _dot(h, wlh_ref[...])
           + bl_ref[...])
    zs = (srow_ref[:, 0:1] * wps_ref[0:1, :]
          + srow_ref[:, 1:2] * wps_ref[1:2, :]
          + srow_ref[:, 2:3] * wps_ref[2:3, :])
    z = _elu(_dot(xgn, wpx_ref[...]) + zs + _dot(h, wph_ref[...])
             + bp1_ref[...])
    z = _elu(_dot(z, wp2_ref[...]) + bp2_ref[...])
    o_ref[...] = _bn(z, g2_ref[...], be2_ref[...])


def _k4(agg, h, srow, W_lin, b_lin, W_post1, b_post1, W_post2, b_post2,
        g2, be2):
    return pl.pallas_call(
        _k4_body,
        out_shape=jax.ShapeDtypeStruct((N, D), jnp.float32),
    )(agg, h, srow, W_lin[:2 * D], W_lin[2 * D:], b_lin[None, :],
      W_post1[:D], W_post1[D:D + SD], W_post1[D + SD:], b_post1[None, :],
      W_post2, b_post2[None, :], g2[None, :], be2[None, :])


def kernel(x, batch, original_coords, step_count, num_layer,
           W_pre1, b_pre1, W_pre2, b_pre2, g1, be1,
           W_s, W_h, b_h, W_lin, b_lin,
           W_post1, b_post1, W_post2, b_post2, g2, be2):
    h, h_l, srow = _k1(x, W_pre1, b_pre1, W_pre2, b_pre2, g1, be1,
                       W_h, b_h, W_s)
    srow_pad = jnp.pad(srow, ((0, NPAD - N), (0, 0)))
    scol_pad = srow_pad.T
    nn_idx = _k2(srow_pad, scol_pad)
    idx48 = jnp.pad(nn_idx, ((0, 0), (0, KP - K)))
    h_l_pad = jnp.pad(h_l, ((0, NPAD - N), (0, 0)))
    s_l = srow[:, :SD]
    s_nbr = s_l[nn_idx[:N]]
    d2_nn = jnp.sum((s_l[:, None, :] - s_nbr) ** 2, axis=-1)
    p = jnp.exp(-d2_nn)
    p48 = jnp.pad(p, ((0, NPAD - N), (0, KP - K)))
    agg = _k3(idx48, p48, h_l_pad)[:N]
    return _k4(agg, h, srow, W_lin, b_lin, W_post1, b_post1,
               W_post2, b_post2, g2, be2)
